# Initial kernel scaffold; baseline (speedup 1.0000x reference)
import jax
import jax.numpy as jnp
from jax.experimental import pallas as pl


def kernel(x, W1, b1, g1, be1, W2, b2, g2, be2, W3, b3, g3, be3, W4, b4, g4, be4, W5, b5, g5, be5, W6, b6, g6, be6, W7, b7):
    raise NotImplementedError("write your pallas kernel here")


# speedup vs baseline: 9.4762x; 9.4762x over previous
import functools

import jax
import jax.numpy as jnp
from jax import lax
from jax.experimental import pallas as pl
from jax.experimental.pallas import tpu as pltpu
from jax.experimental.pallas import tpu_sc as plsc

N = 4096
K = 20
BLK = 256
NBLK = N // BLK
F = 64
FP = 128
NEDGE = N * K


def _bdot(a, b):
    return jnp.dot(a.astype(jnp.bfloat16), b.astype(jnp.bfloat16),
                   preferred_element_type=jnp.float32)


def _dist_topk_body(norm, nd, y_ref, yb_ref, s_ref, s2_ref, g_ref, be_ref,
                    idx_ref, xnp_ref):
    b = pl.program_id(0)
    x = y_ref[...]
    xb = yb_ref[...]
    if norm:
        cnt = jnp.float32(NEDGE)
        mu = s_ref[...] / cnt
        var = s2_ref[...] / cnt - mu * mu
        scale = lax.rsqrt(var + 1e-5) * g_ref[...]
        x = (x - mu) * scale + be_ref[...]
        xb = (xb - mu) * scale + be_ref[...]
    xnp_ref[:, :nd] = xb
    xnp_ref[:, nd:] = jnp.zeros((BLK, FP - nd), jnp.float32)
    sq = jnp.sum(x * x, axis=1)[None, :]
    sqb = jnp.sum(xb * xb, axis=1)[:, None]
    dots = lax.dot_general(xb.astype(jnp.bfloat16), x.astype(jnp.bfloat16),
                           (((1,), (1,)), ((), ())),
                           preferred_element_type=jnp.float32)
    dist = sqb + sq - 2.0 * dots
    cols = lax.broadcasted_iota(jnp.int32, (BLK, N), 1)
    rows = lax.broadcasted_iota(jnp.int32, (BLK, N), 0) + b * BLK
    dist = jnp.where(cols == rows, dist + 1e10, dist)
    big_i = jnp.int32(2 ** 30)
    for k in range(K):
        mn = jnp.min(dist, axis=1, keepdims=True)
        cand = jnp.where(dist == mn, cols, big_i)
        am = jnp.min(cand, axis=1)
        idx_ref[:, k] = am
        dist = jnp.where(cols == am[:, None], jnp.float32(3e38), dist)


def _dist_topk(y, s, s2, g, be, norm, interpret=False):
    nd = y.shape[1]
    return pl.pallas_call(
        functools.partial(_dist_topk_body, norm, nd),
        grid=(NBLK,),
        in_specs=[
            pl.BlockSpec((N, nd), lambda b: (0, 0)),
            pl.BlockSpec((BLK, nd), lambda b: (b, 0)),
            pl.BlockSpec((1, F), lambda b: (0, 0)),
            pl.BlockSpec((1, F), lambda b: (0, 0)),
            pl.BlockSpec((1, F), lambda b: (0, 0)),
            pl.BlockSpec((1, F), lambda b: (0, 0)),
        ],
        out_specs=(
            pl.BlockSpec((BLK, K), lambda b: (b, 0)),
            pl.BlockSpec((BLK, FP), lambda b: (b, 0)),
        ),
        out_shape=(
            jax.ShapeDtypeStruct((N, K), jnp.int32),
            jax.ShapeDtypeStruct((N, FP), jnp.float32),
        ),
        interpret=interpret,
    )(y, y, s, s2, g, be)


_NW = 32
_BPW = NEDGE // _NW
_CH = 128
_NCH = _BPW // _CH


def _gather(xnp, idx_w):
    mesh = plsc.VectorSubcoreMesh(core_axis_name="c", subcore_axis_name="s")

    @functools.partial(
        pl.kernel,
        mesh=mesh,
        out_type=jax.ShapeDtypeStruct((NEDGE, FP), jnp.float32),
        scratch_types=[
            pltpu.VMEM((_BPW,), jnp.int32),
            pltpu.VMEM((_CH, FP), jnp.float32),
            pltpu.SemaphoreType.DMA,
        ],
    )
    def k(x_hbm, idx_hbm, out_hbm, idx_v, rows_v, sem):
        wid = lax.axis_index("s") * 2 + lax.axis_index("c")
        pltpu.sync_copy(idx_hbm.at[wid], idx_v)
        base = wid * _BPW
        for c in range(_NCH):
            pltpu.async_copy(
                x_hbm.at[idx_v.at[pl.ds(c * _CH, _CH)]], rows_v, sem).wait()
            pltpu.sync_copy(rows_v, out_hbm.at[pl.ds(base + c * _CH, _CH)])

    return k(xnp, idx_w)


def _edge_reduce_body(e_ref, xi_ref, wt_ref, wb_ref, b_ref,
                      ymin_ref, s_ref, s2_ref):
    b = pl.program_id(0)
    xi = xi_ref[...]
    wt = wt_ref[...].astype(jnp.bfloat16)
    wb = wb_ref[...].astype(jnp.bfloat16)
    hb = jnp.dot(xi.astype(jnp.bfloat16), wt,
                 preferred_element_type=jnp.float32) + b_ref[...]

    def msg(k):
        diff = e_ref[k] - xi
        return jnp.maximum(
            hb + jnp.dot(diff.astype(jnp.bfloat16), wb,
                         preferred_element_type=jnp.float32), 0.0)

    h = msg(0)
    hmin = h
    s = h
    s2 = h * h
    for k in range(1, K):
        h = msg(k)
        hmin = jnp.minimum(hmin, h)
        s = s + h
        s2 = s2 + h * h
    ymin_ref[...] = hmin
    ps = jnp.sum(s, axis=0, keepdims=True)
    ps2 = jnp.sum(s2, axis=0, keepdims=True)

    @pl.when(b == 0)
    def _():
        s_ref[...] = jnp.zeros_like(s_ref)
        s2_ref[...] = jnp.zeros_like(s2_ref)

    s_ref[...] += ps
    s2_ref[...] += ps2


def _edge_reduce(edges3, xnp, wt, wb, bvec, interpret=False):
    return pl.pallas_call(
        _edge_reduce_body,
        grid=(NBLK,),
        in_specs=[
            pl.BlockSpec((K, BLK, FP), lambda b: (0, b, 0)),
            pl.BlockSpec((BLK, FP), lambda b: (b, 0)),
            pl.BlockSpec((FP, F), lambda b: (0, 0)),
            pl.BlockSpec((FP, F), lambda b: (0, 0)),
            pl.BlockSpec((1, F), lambda b: (0, 0)),
        ],
        out_specs=(
            pl.BlockSpec((BLK, F), lambda b: (b, 0)),
            pl.BlockSpec((1, F), lambda b: (0, 0)),
            pl.BlockSpec((1, F), lambda b: (0, 0)),
        ),
        out_shape=(
            jax.ShapeDtypeStruct((N, F), jnp.float32),
            jax.ShapeDtypeStruct((1, F), jnp.float32),
            jax.ShapeDtypeStruct((1, F), jnp.float32),
        ),
        interpret=interpret,
    )(edges3, xnp, wt, wb, bvec)


def _head_body(x1_ref, x2_ref, y3_ref, s3_ref, s23_ref, g3_ref, be3_ref,
               w4a_ref, w4b_ref, w4c_ref, b4_ref, g4_ref, be4_ref,
               w5_ref, b5_ref, g5_ref, be5_ref,
               w6_ref, b6_ref, g6_ref, be6_ref,
               w7_ref, b7_ref, out_ref):
    cnt = jnp.float32(NEDGE)
    mu3 = s3_ref[...] / cnt
    var3 = s23_ref[...] / cnt - mu3 * mu3
    x3 = (y3_ref[...] - mu3) * lax.rsqrt(var3 + 1e-5) * g3_ref[...] \
        + be3_ref[...]

    def mlp(h, g, be):
        h = jnp.maximum(h, 0.0)
        mu = jnp.mean(h, axis=0, keepdims=True)
        var = jnp.mean(h * h, axis=0, keepdims=True) - mu * mu
        return (h - mu) * lax.rsqrt(var + 1e-5) * g + be

    h = (_bdot(x1_ref[...], w4a_ref[...]) + _bdot(x2_ref[...], w4b_ref[...])
         + _bdot(x3, w4c_ref[...]) + b4_ref[...])
    h = mlp(h, g4_ref[...], be4_ref[...])
    h = mlp(_bdot(h, w5_ref[...]) + b5_ref[...], g5_ref[...], be5_ref[...])
    h = mlp(_bdot(h, w6_ref[...]) + b6_ref[...], g6_ref[...], be6_ref[...])
    o = _bdot(h, w7_ref[...]) + b7_ref[...]
    mx = jnp.max(o, axis=1, keepdims=True)
    z = o - mx
    lse = jnp.log(jnp.sum(jnp.exp(z), axis=1, keepdims=True))
    out_ref[...] = z - lse


def _head(x1, x2, y3, s3, s23, g3, be3, w4a, w4b, w4c, b4, g4, be4,
          w5, b5, g5, be5, w6, b6, g6, be6, w7, b7, interpret=False):
    return pl.pallas_call(
        _head_body,
        out_shape=jax.ShapeDtypeStruct((N, 40), jnp.float32),
        interpret=interpret,
    )(x1, x2, y3, s3, s23, g3, be3, w4a, w4b, w4c, b4, g4, be4,
      w5, b5, g5, be5, w6, b6, g6, be6, w7, b7)


def _row(v):
    return v.reshape(1, -1)


def _pad_w(w):
    return jnp.concatenate(
        [w, jnp.zeros((FP - w.shape[0], F), jnp.float32)], axis=0)


def _edge_layer(y_prev, s_prev, s2_prev, g_prev, be_prev, W, bvec, norm,
                interpret=False, gather_fn=None):
    din = W.shape[0] // 2
    wt, wb = _pad_w(W[:din]), _pad_w(W[din:])
    idx, xnp = _dist_topk(y_prev, s_prev, s2_prev, g_prev, be_prev, norm,
                          interpret=interpret)
    idx_w = idx.T.reshape(_NW, _BPW)
    if gather_fn is None:
        edges = _gather(xnp, idx_w)
    else:
        edges = gather_fn(xnp, idx_w)
    edges3 = edges.reshape(K, N, FP)
    ymin, s, s2 = _edge_reduce(edges3, xnp, wt, wb, _row(bvec),
                               interpret=interpret)
    return xnp, ymin, s, s2


def kernel(x, W1, b1, g1, be1, W2, b2, g2, be2, W3, b3, g3, be3,
           W4, b4, g4, be4, W5, b5, g5, be5, W6, b6, g6, be6, W7, b7,
           interpret=False, gather_fn=None):
    xpad = jnp.concatenate(
        [x, jnp.zeros((N, 5), jnp.float32)], axis=1)
    zs = jnp.zeros((1, F), jnp.float32)
    _, y1, s1, s21 = _edge_layer(xpad, zs, zs, zs, zs, W1, b1, norm=False,
                                 interpret=interpret, gather_fn=gather_fn)
    xnp2, y2, s2_, s22 = _edge_layer(y1, s1, s21, _row(g1), _row(be1),
                                     W2, b2, norm=True, interpret=interpret,
                                     gather_fn=gather_fn)
    xnp3, y3, s3, s23 = _edge_layer(y2, s2_, s22, _row(g2), _row(be2),
                                    W3, b3, norm=True, interpret=interpret,
                                    gather_fn=gather_fn)
    x1 = xnp2[:, :F]
    x2 = xnp3[:, :F]
    w4a, w4b, w4c = W4[:64], W4[64:128], W4[128:]
    return _head(x1, x2, y3, s3, s23, _row(g3), _row(be3),
                 w4a, w4b, w4c, _row(b4), _row(g4), _row(be4),
                 W5, _row(b5), _row(g5), _row(be5),
                 W6, _row(b6), _row(g6), _row(be6),
                 W7, _row(b7), interpret=interpret)
